# baseline (speedup 1.0000x reference)
import functools

import jax
import jax.numpy as jnp
from jax import lax
from jax.experimental import pallas as pl
from jax.experimental.pallas import tpu as pltpu
from jax.experimental.pallas import tpu_sc as plsc

_DIM = 64
_SCALE = 8.0
_NC, _NS = 2, 16
_NW = _NC * _NS
_CHUNK = 128
_LANES = 16
_GROUPS = _DIM // _LANES


def _scale_rows(src, dst, sb, db):
  def body(r, _):
    for j in range(_GROUPS):
      v = src[sb, r, pl.ds(j * _LANES, _LANES)]
      dst[db, r, pl.ds(j * _LANES, _LANES)] = v * _SCALE
    return 0

  lax.fori_loop(0, _CHUNK, body, 0, unroll=4)


def _make_kernel(n_idx):
  b_per_w = n_idx // _NW
  n_chunk = b_per_w // _CHUNK
  assert n_chunk % 2 == 0 and n_chunk >= 4

  mesh = plsc.VectorSubcoreMesh(core_axis_name="c", subcore_axis_name="s")

  @functools.partial(
      pl.kernel,
      out_type=jax.ShapeDtypeStruct((n_idx, _DIM), jnp.float32),
      mesh=mesh,
      compiler_params=pltpu.CompilerParams(use_tc_tiling_on_sc=False),
      scratch_types=[
          pltpu.VMEM((n_chunk, _CHUNK), jnp.int32),
          pltpu.VMEM((2, _CHUNK, _DIM), jnp.float32),
          pltpu.VMEM((2, _CHUNK, _DIM), jnp.float32),
          pltpu.SemaphoreType.DMA,
          pltpu.SemaphoreType.DMA,
          pltpu.SemaphoreType.DMA,
          pltpu.SemaphoreType.DMA,
      ],
  )
  def emb_kernel(x_hbm, lut_hbm, out_hbm, idx_v, gbuf, sbuf,
                 gsem0, gsem1, ssem0, ssem1):
    wid = lax.axis_index("s") * _NC + lax.axis_index("c")
    row0 = wid * n_chunk
    base = wid * b_per_w

    pltpu.sync_copy(x_hbm.at[pl.ds(row0, n_chunk), :], idx_v)

    gsems = (gsem0, gsem1)
    ssems = (ssem0, ssem1)

    def gather_start(i, b):
      pltpu.async_copy(lut_hbm.at[idx_v.at[i]], gbuf.at[b], gsems[b])

    def gather_wait(i, b):
      pltpu.make_async_copy(lut_hbm.at[idx_v.at[i]], gbuf.at[b],
                            gsems[b]).wait()

    def store_start(i, b):
      pltpu.async_copy(sbuf.at[b], out_hbm.at[pl.ds(base + i * _CHUNK,
                                                    _CHUNK), :], ssems[b])

    def store_wait(i, b):
      pltpu.make_async_copy(sbuf.at[b], out_hbm.at[pl.ds(base + i * _CHUNK,
                                                         _CHUNK), :],
                            ssems[b]).wait()

    gather_start(0, 0)
    gather_start(1, 1)

    for b in (0, 1):
      gather_wait(b, b)
      _scale_rows(gbuf, sbuf, b, b)
      gather_start(b + 2, b)
      store_start(b, b)

    def body(k, _):
      i0 = 2 * k
      for b in (0, 1):
        i = i0 + b
        gather_wait(i, b)
        store_wait(i - 2, b)
        _scale_rows(gbuf, sbuf, b, b)
        gather_start(i + 2, b)
        store_start(i, b)
      return 0

    lax.fori_loop(1, n_chunk // 2 - 1, body, 0)

    for b in (0, 1):
      i = n_chunk - 2 + b
      gather_wait(i, b)
      store_wait(i - 2, b)
      _scale_rows(gbuf, sbuf, b, b)
      store_start(i, b)

    for b in (0, 1):
      store_wait(n_chunk - 2 + b, b)

  return emb_kernel


def kernel(x, lut):
  orig_shape = x.shape
  n_idx = x.size
  x2d = x.reshape(n_idx // _CHUNK, _CHUNK).astype(jnp.int32)
  out = _make_kernel(n_idx)(x2d, lut)
  return out.reshape(*orig_shape, _DIM)


# speedup vs baseline: 1.8909x; 1.8909x over previous
import functools

import jax
import jax.numpy as jnp
from jax import lax
from jax.experimental import pallas as pl
from jax.experimental.pallas import tpu as pltpu
from jax.experimental.pallas import tpu_sc as plsc

_DIM = 64
_SCALE = 8.0
_NC, _NS = 2, 16
_NW = _NC * _NS
_CHUNK = 128
_LANES = 16
_GROUPS = _DIM // _LANES
_NBUF = 4


def _make_kernel(n_idx):
  b_per_w = n_idx // _NW
  n_chunk = b_per_w // _CHUNK
  n_rows = b_per_w // _CHUNK
  assert n_chunk % _NBUF == 0 and n_chunk >= 3 * _NBUF

  mesh = plsc.VectorSubcoreMesh(core_axis_name="c", subcore_axis_name="s")

  @functools.partial(
      pl.kernel,
      out_type=jax.ShapeDtypeStruct((n_idx, _DIM), jnp.float32),
      mesh=mesh,
      scratch_types=[
          pltpu.VMEM((n_rows, _CHUNK), jnp.int32),
          pltpu.VMEM((_NBUF, _CHUNK, _DIM), jnp.float32),
          pltpu.SemaphoreType.DMA,
          pltpu.SemaphoreType.DMA,
          pltpu.SemaphoreType.DMA,
          pltpu.SemaphoreType.DMA,
          pltpu.SemaphoreType.DMA,
          pltpu.SemaphoreType.DMA,
      ],
  )
  def emb_kernel(x_hbm, lut_hbm, out_hbm, idx_v, rows, gsem0, gsem1,
                 ssem0, ssem1, ssem2, ssem3):
    wid = lax.axis_index("s") * _NC + lax.axis_index("c")
    base = wid * b_per_w
    gsems = (gsem0, gsem1)
    ssems = (ssem0, ssem1, ssem2, ssem3)

    pltpu.sync_copy(x_hbm.at[pl.ds(wid * n_rows, n_rows), :], idx_v)

    def gather_rows(i, b, p):
      def issue(g, _):
        vec = idx_v[i, pl.ds(g * _LANES, _LANES)]
        for k in range(_LANES):
          r = vec[k]
          pltpu.async_copy(
              lut_hbm.at[pl.ds(r, 1), :],
              rows.at[b, pl.ds(g * _LANES + k, 1), :], gsems[p])
        return 0

      lax.fori_loop(0, _CHUNK // _LANES, issue, 0)

    def gather_drain(b, p):
      pltpu.make_async_copy(lut_hbm.at[pl.ds(0, _CHUNK), :], rows.at[b],
                            gsems[p]).wait()

    def scale_chunk(b):
      def body(r, _):
        for j in range(_GROUPS):
          sl = pl.ds(j * _LANES, _LANES)
          rows[b, r, sl] = rows[b, r, sl] * _SCALE
        return 0

      lax.fori_loop(0, _CHUNK, body, 0, unroll=8)

    def store_start(i, b):
      pltpu.async_copy(rows.at[b],
                       out_hbm.at[pl.ds(base + i * _CHUNK, _CHUNK), :],
                       ssems[b])

    def store_wait(i, b):
      pltpu.make_async_copy(rows.at[b],
                            out_hbm.at[pl.ds(base + i * _CHUNK, _CHUNK), :],
                            ssems[b]).wait()

    def step(i, b, p, *, wait_store=True, do_next=True):
      bn = (b + 1) % _NBUF
      if do_next:
        if wait_store:
          store_wait(i - 3, bn)
        gather_rows(i + 1, bn, (p + 1) % 2)
      gather_drain(b, p)
      scale_chunk(b)
      store_start(i, b)

    gather_rows(0, 0, 0)

    for i in range(3):
      step(i, i % _NBUF, i % 2, wait_store=False)

    def body(k, _):
      i0 = _NBUF * k
      for d in range(_NBUF):
        i = i0 + d - 1
        step(i, (d - 1) % _NBUF, (d - 1) % 2)
      return 0

    lax.fori_loop(1, n_chunk // _NBUF - 1, body, 0)

    for i in range(n_chunk - 5, n_chunk - 1):
      step(i, i % _NBUF, i % 2)
    step(n_chunk - 1, (n_chunk - 1) % _NBUF, (n_chunk - 1) % 2,
         do_next=False)

    for i in range(n_chunk - 4, n_chunk):
      store_wait(i, i % _NBUF)

  return emb_kernel


def kernel(x, lut):
  orig_shape = x.shape
  n_idx = x.size
  x2d = x.reshape(n_idx // _CHUNK, _CHUNK).astype(jnp.int32)
  out = _make_kernel(n_idx)(x2d, lut)
  return out.reshape(*orig_shape, _DIM)
